# baseline (speedup 1.0000x reference)
import functools
import hashlib

import jax
import jax.numpy as jnp
import numpy as np
from jax import lax
from jax.experimental import pallas as pl
from jax.experimental.pallas import tpu as pltpu
from jax.experimental.pallas import tpu_sc as plsc

NUM_BUCKETS = 100000
EMBED_DIM = 32
VOCAB = 1000000

BATCH, SEQ = 4096, 50
NC, NS = 2, 16
NW = NC * NS
ROWS_PER_W = BATCH // NW
TOK_PER_W = ROWS_PER_W * SEQ
LANES = 16

CHUNK = 80
NCHUNK = TOK_PER_W // CHUNK
S1G = 20
NS1G = NCHUNK // S1G
GROUPC = 10
GROUP_TOK = GROUPC * CHUNK
GROUP_ROWS = GROUP_TOK // SEQ
NGROUPS = NCHUNK // GROUPC

_SC_PARAMS = pltpu.CompilerParams(use_tc_tiling_on_sc=False,
                                  needs_layout_passes=False)


def _build_lut() -> np.ndarray:
    lut = np.empty((VOCAB,), dtype=np.int32)
    for t in range(VOCAB):
        h = hashlib.md5(str(t).encode()).hexdigest()
        lut[t] = int(h, 16) % NUM_BUCKETS
    return lut


_LUT = _build_lut()


def _make_hash_kernel():
    mesh = plsc.VectorSubcoreMesh(core_axis_name="c", subcore_axis_name="s")

    @functools.partial(
        pl.kernel,
        mesh=mesh,
        compiler_params=_SC_PARAMS,
        out_type=jax.ShapeDtypeStruct((NW, TOK_PER_W), jnp.int32),
        scratch_types=[
            pltpu.VMEM((ROWS_PER_W, SEQ), jnp.int32),
            pltpu.VMEM((TOK_PER_W,), jnp.int32),
            pltpu.VMEM((TOK_PER_W,), jnp.int32),
            pltpu.SemaphoreType.DMA,
        ],
    )
    def ka(ids_hbm, lut_hbm, hashed_hbm, ids_v, flat_v, hashed_v, sem1):
        wid = lax.axis_index("s") * NC + lax.axis_index("c")
        b0 = wid * ROWS_PER_W
        lane = lax.iota(jnp.int32, LANES)

        pltpu.sync_copy(ids_hbm.at[pl.ds(b0, ROWS_PER_W)], ids_v)

        @plsc.parallel_loop(0, TOK_PER_W // LANES, 1, unroll=4)
        def repack(i):
            o = i * LANES
            t = o + lane
            flat_v[pl.ds(o, LANES)] = plsc.load_gather(
                ids_v, [t // SEQ, lax.rem(t, SEQ)])

        def fire_lut(j, c):
            pltpu.async_copy(lut_hbm.at[flat_v.at[pl.ds(j * CHUNK, CHUNK)]],
                             hashed_v.at[pl.ds(j * CHUNK, CHUNK)], sem1)
            return c

        def wait_lut_window():
            pltpu.make_async_copy(lut_hbm.at[pl.ds(0, S1G * CHUNK)],
                                  hashed_v.at[pl.ds(0, S1G * CHUNK)],
                                  sem1).wait()

        def s1_window(w, c):
            lax.fori_loop(w * S1G, (w + 1) * S1G, fire_lut, 0)

            @pl.when(w >= 1)
            def _():
                wait_lut_window()
            return c
        lax.fori_loop(0, NS1G, s1_window, 0)
        wait_lut_window()

        pltpu.sync_copy(hashed_v, hashed_hbm.at[wid])

    return ka


def _make_lookup_kernel():
    mesh = plsc.VectorSubcoreMesh(core_axis_name="c", subcore_axis_name="s")

    @functools.partial(
        pl.kernel,
        mesh=mesh,
        compiler_params=_SC_PARAMS,
        out_type=jax.ShapeDtypeStruct((SEQ, EMBED_DIM // 8, NW, 8, BATCH // NW),
                                      jnp.float32),
        scratch_types=[
            pltpu.VMEM((TOK_PER_W,), jnp.int32),
            pltpu.VMEM((2, GROUP_TOK, EMBED_DIM), jnp.float32),
            pltpu.VMEM((2, SEQ, EMBED_DIM // 8, 8, GROUP_ROWS), jnp.float32),
            pltpu.SemaphoreType.DMA,
            pltpu.SemaphoreType.DMA,
        ],
    )
    def kb(hashed_hbm, table_hbm, out_hbm, hashed_v, rows_v, tr_v,
           sem2, sem3):
        wid = lax.axis_index("s") * NC + lax.axis_index("c")
        lane = lax.iota(jnp.int32, LANES)
        lane50 = lane * SEQ

        pltpu.sync_copy(hashed_hbm.at[wid], hashed_v)

        def fire_group(g):
            vbuf = lax.rem(g, 2)

            def fire_rows(j, c2):
                pltpu.async_copy(
                    table_hbm.at[hashed_v.at[pl.ds((g * GROUPC + j) * CHUNK,
                                                   CHUNK)]],
                    rows_v.at[vbuf].at[pl.ds(j * CHUNK, CHUNK)], sem2)
                return c2
            lax.fori_loop(0, GROUPC, fire_rows, 0)

        fire_group(0)

        def group(g, c):
            vbuf = lax.rem(g, 2)

            @pl.when(g + 1 < NGROUPS)
            def _():
                fire_group(g + 1)

            pltpu.make_async_copy(
                table_hbm.at[pl.ds(0, GROUP_TOK)], rows_v.at[vbuf],
                sem2).wait()

            @pl.when(g >= 2)
            def _():
                pltpu.make_async_copy(
                    out_hbm.at[:, :, 0, :, pl.ds(0, GROUP_ROWS)],
                    tr_v.at[vbuf], sem3).wait()

            @plsc.parallel_loop(0, SEQ, 1, unroll=2)
            def trans_row(s):
                rowvec = lane50 + s
                src = rows_v.at[vbuf]
                for d in range(EMBED_DIM):
                    tr_v[vbuf, s, d // 8, d % 8, :] = plsc.load_gather(
                        src, [rowvec, jnp.full((LANES,), d, jnp.int32)])

            pltpu.async_copy(
                tr_v.at[vbuf],
                out_hbm.at[:, :, wid, :, pl.ds(g * GROUP_ROWS, GROUP_ROWS)],
                sem3)
            return c
        lax.fori_loop(0, NGROUPS, group, 0)

        pltpu.make_async_copy(out_hbm.at[:, :, 0, :, pl.ds(0, GROUP_ROWS)],
                              tr_v.at[0], sem3).wait()
        pltpu.make_async_copy(out_hbm.at[:, :, 0, :, pl.ds(0, GROUP_ROWS)],
                              tr_v.at[1], sem3).wait()

    return kb


_sc_hash = _make_hash_kernel()
_sc_lookup = _make_lookup_kernel()


def kernel(input_ids, table):
    hashed = _sc_hash(input_ids, jnp.asarray(_LUT))
    out_t = _sc_lookup(hashed, table)
    return jnp.transpose(out_t, (2, 4, 0, 1, 3)).reshape(BATCH, SEQ, EMBED_DIM)


# speedup vs baseline: 1.1074x; 1.1074x over previous
import functools
import hashlib

import jax
import jax.numpy as jnp
import numpy as np
from jax import lax
from jax.experimental import pallas as pl
from jax.experimental.pallas import tpu as pltpu
from jax.experimental.pallas import tpu_sc as plsc

NUM_BUCKETS = 100000
EMBED_DIM = 32
VOCAB = 1000000

BATCH, SEQ = 4096, 50
NC, NS = 2, 16
NW = NC * NS
ROWS_PER_W = BATCH // NW
TOK_PER_W = ROWS_PER_W * SEQ
LANES = 16

CHUNK = 80
NCHUNK = TOK_PER_W // CHUNK
S1G = 10
NS1G = NCHUNK // S1G
GROUPC = 10
GROUP_TOK = GROUPC * CHUNK
GROUP_ROWS = GROUP_TOK // SEQ
NGROUPS = NCHUNK // GROUPC


def _build_lut() -> np.ndarray:
    lut = np.empty((VOCAB,), dtype=np.int32)
    for t in range(VOCAB):
        h = hashlib.md5(str(t).encode()).hexdigest()
        lut[t] = int(h, 16) % NUM_BUCKETS
    return lut


_LUT = _build_lut()


def _make_sc_kernel():
    mesh = plsc.VectorSubcoreMesh(core_axis_name="c", subcore_axis_name="s")

    @functools.partial(
        pl.kernel,
        mesh=mesh,
        compiler_params=pltpu.CompilerParams(use_tc_tiling_on_sc=False,
                                             needs_layout_passes=False),
        out_type=jax.ShapeDtypeStruct((SEQ, EMBED_DIM // 8, NW, 8, BATCH // NW),
                                      jnp.float32),
        scratch_types=[
            pltpu.VMEM((ROWS_PER_W, SEQ), jnp.int32),
            pltpu.VMEM((TOK_PER_W,), jnp.int32),
            pltpu.VMEM((TOK_PER_W,), jnp.int32),
            pltpu.VMEM((2, GROUP_TOK, EMBED_DIM), jnp.float32),
            pltpu.VMEM((2, SEQ, EMBED_DIM // 8, 8, GROUP_ROWS), jnp.float32),
            pltpu.SemaphoreType.DMA,
            pltpu.SemaphoreType.DMA,
            pltpu.SemaphoreType.DMA,
        ],
    )
    def k(ids_hbm, lut_hbm, table_hbm, out_hbm, ids_v, flat_v, hashed_v,
          rows_v, tr_v, sem1, sem2, sem3):
        wid = lax.axis_index("s") * NC + lax.axis_index("c")
        b0 = wid * ROWS_PER_W
        lane = lax.iota(jnp.int32, LANES)

        pltpu.sync_copy(ids_hbm.at[pl.ds(b0, ROWS_PER_W)], ids_v)

        @plsc.parallel_loop(0, TOK_PER_W // LANES, 1, unroll=4)
        def repack(i):
            o = i * LANES
            t = o + lane
            flat_v[pl.ds(o, LANES)] = plsc.load_gather(
                ids_v, [t // SEQ, lax.rem(t, SEQ)])

        def fire_lut(j, c):
            pltpu.async_copy(lut_hbm.at[flat_v.at[pl.ds(j * CHUNK, CHUNK)]],
                             hashed_v.at[pl.ds(j * CHUNK, CHUNK)], sem1)
            return c

        def fire_lut_window(w):
            lax.fori_loop(w * S1G, (w + 1) * S1G, fire_lut, 0)

        def wait_lut_window():
            pltpu.make_async_copy(lut_hbm.at[pl.ds(0, S1G * CHUNK)],
                                  hashed_v.at[pl.ds(0, S1G * CHUNK)],
                                  sem1).wait()

        fire_lut_window(0)
        wait_lut_window()
        fire_lut_window(1)

        lane50 = lane * SEQ

        def fire_group(g):
            vbuf = lax.rem(g, 2)

            def fire_rows(j, c2):
                pltpu.async_copy(
                    table_hbm.at[hashed_v.at[pl.ds((g * GROUPC + j) * CHUNK,
                                                   CHUNK)]],
                    rows_v.at[vbuf].at[pl.ds(j * CHUNK, CHUNK)], sem2)
                return c2
            lax.fori_loop(0, GROUPC, fire_rows, 0)

        fire_group(0)

        def group(g, c):
            vbuf = lax.rem(g, 2)

            @pl.when(g + 1 < NGROUPS)
            def _():
                wait_lut_window()

                @pl.when(g + 2 < NS1G)
                def _():
                    fire_lut_window(g + 2)

                fire_group(g + 1)

            pltpu.make_async_copy(
                table_hbm.at[pl.ds(0, GROUP_TOK)], rows_v.at[vbuf],
                sem2).wait()

            @pl.when(g >= 2)
            def _():
                pltpu.make_async_copy(out_hbm.at[:, :, 0, :, pl.ds(0, GROUP_ROWS)],
                                      tr_v.at[vbuf], sem3).wait()

            @plsc.parallel_loop(0, SEQ, 1, unroll=2)
            def trans_row(s):
                rowvec = lane50 + s
                src = rows_v.at[vbuf]
                for d in range(EMBED_DIM):
                    tr_v[vbuf, s, d // 8, d % 8, :] = plsc.load_gather(
                        src, [rowvec, jnp.full((LANES,), d, jnp.int32)])

            pltpu.async_copy(
                tr_v.at[vbuf],
                out_hbm.at[:, :, wid, :, pl.ds(g * GROUP_ROWS, GROUP_ROWS)],
                sem3)
            return c
        lax.fori_loop(0, NGROUPS, group, 0)

        pltpu.make_async_copy(out_hbm.at[:, :, 0, :, pl.ds(0, GROUP_ROWS)],
                              tr_v.at[0], sem3).wait()
        pltpu.make_async_copy(out_hbm.at[:, :, 0, :, pl.ds(0, GROUP_ROWS)],
                              tr_v.at[1], sem3).wait()

    return k


_sc_lookup = _make_sc_kernel()


def kernel(input_ids, table):
    out_t = _sc_lookup(input_ids, jnp.asarray(_LUT), table)
    return jnp.transpose(out_t, (2, 4, 0, 1, 3)).reshape(BATCH, SEQ, EMBED_DIM)
